# baseline (speedup 1.0000x reference)
import jax
import jax.numpy as jnp
from jax.experimental import pallas as pl
from jax.experimental.pallas import tpu as pltpu


def _im2col_nhwc(x, kh, kw, stride):
    n, h, w, c = x.shape
    ho = (h - kh) // stride + 1
    wo = (w - kw) // stride + 1
    taps = []
    for i in range(kh):
        for j in range(kw):
            taps.append(x[:, i:i + stride * ho:stride, j:j + stride * wo:stride, :])
    cols = jnp.stack(taps, axis=3)
    return cols.reshape(n, ho * wo, kh * kw * c)


def _mm_bias_relu_kernel(x_ref, w_ref, b_ref, o_ref):
    acc = jnp.dot(x_ref[...], w_ref[...], preferred_element_type=jnp.float32)
    o_ref[...] = jnp.maximum(acc + b_ref[...], 0.0).astype(o_ref.dtype)


def _matmul_bias_relu(x, w, b, bm):
    m, k = x.shape
    n = w.shape[1]
    assert m % bm == 0, (m, bm)
    return pl.pallas_call(
        _mm_bias_relu_kernel,
        out_shape=jax.ShapeDtypeStruct((m, n), jnp.bfloat16),
        grid=(m // bm,),
        in_specs=[pl.BlockSpec((bm, k), lambda i: (i, 0)),
                  pl.BlockSpec((k, n), lambda i: (0, 0)),
                  pl.BlockSpec((1, n), lambda i: (0, 0))],
        out_specs=pl.BlockSpec((bm, n), lambda i: (i, 0)),
        compiler_params=pltpu.CompilerParams(
            dimension_semantics=("parallel",)),
    )(x, w, b)


def _fc_head_kernel(a_ref, w1_ref, b1_ref, w2_ref, b2_ref, w3_ref, b3_ref,
                    o_ref):
    h = jnp.dot(a_ref[...], w1_ref[...], preferred_element_type=jnp.float32)
    h = jnp.maximum(h + b1_ref[...], 0.0).astype(jnp.bfloat16)
    h = jnp.dot(h, w2_ref[...], preferred_element_type=jnp.float32)
    h = jnp.maximum(h + b2_ref[...], 0.0).astype(jnp.bfloat16)
    q = jnp.dot(h, w3_ref[...], preferred_element_type=jnp.float32)
    o_ref[...] = q + b3_ref[...]


def _fc_head(a, fc1_w, fc1_b, fc2_w, fc2_b, fc3_w, fc3_b, bm):
    m, k = a.shape
    n_pad = fc3_w.shape[1]
    assert m % bm == 0
    consts = [fc1_w, fc1_b, fc2_w, fc2_b, fc3_w, fc3_b]
    in_specs = [pl.BlockSpec((bm, k), lambda i: (i, 0))]
    in_specs += [pl.BlockSpec(c.shape, lambda i: (0,) * c.ndim) for c in consts]
    return pl.pallas_call(
        _fc_head_kernel,
        out_shape=jax.ShapeDtypeStruct((m, n_pad), jnp.float32),
        grid=(m // bm,),
        in_specs=in_specs,
        out_specs=pl.BlockSpec((bm, n_pad), lambda i: (i, 0)),
        compiler_params=pltpu.CompilerParams(
            dimension_semantics=("parallel",)),
    )(a, *consts)


def kernel(x, conv1_w, conv1_b, conv2_s, conv2_w, conv2_b,
           conv3_s, conv3_w, conv3_b, conv4_s, conv4_w, conv4_b,
           fc1_w, fc1_b, fc2_w, fc2_b, fc3_w, fc3_b):
    n = x.shape[0]
    xh = jnp.transpose(x, (0, 2, 3, 1)).astype(jnp.bfloat16)

    p1 = _im2col_nhwc(xh, 8, 8, 4).reshape(n * 400, 256)
    a1 = _matmul_bias_relu(p1, conv1_w, conv1_b, bm=2048)

    p2 = _im2col_nhwc(a1.reshape(n, 20, 20, 32), 4, 4, 2).reshape(n * 81, 512)
    a2 = _matmul_bias_relu(p2, conv2_w.reshape(512, 64), conv2_b, bm=1296)

    p3 = _im2col_nhwc(a2.reshape(n, 9, 9, 64), 3, 3, 1).reshape(n * 49, 576)
    a3 = _matmul_bias_relu(p3, conv3_w.reshape(576, 64), conv3_b, bm=1568)

    p4 = _im2col_nhwc(a3.reshape(n, 7, 7, 64), 3, 3, 1).reshape(n * 25, 576)
    a4 = _matmul_bias_relu(p4, conv4_w.reshape(576, 32), conv4_b, bm=800)

    feats = a4.reshape(n, 800)
    q = _fc_head(feats, fc1_w.reshape(800, 512), fc1_b,
                 fc2_w, fc2_b, fc3_w, fc3_b, bm=128)
    return q[:, :6]


# speedup vs baseline: 24.8416x; 24.8416x over previous
import jax
import jax.numpy as jnp
from jax.experimental import pallas as pl
from jax.experimental.pallas import tpu as pltpu


def _im2col_nhwc(x, kh, kw, stride):
    n, h, w, c = x.shape
    ho = (h - kh) // stride + 1
    wo = (w - kw) // stride + 1
    taps = []
    for i in range(kh):
        for j in range(kw):
            taps.append(x[:, i:i + stride * ho:stride, j:j + stride * wo:stride, :])
    cols = jnp.stack(taps, axis=3)
    return cols.reshape(n, ho * wo, kh * kw * c)


def _conv1_kernel(p_ref, w_ref, b_ref, o_ref):
    bb, m1, k1 = p_ref.shape
    acc = jnp.dot(p_ref[...].reshape(bb * m1, k1), w_ref[...],
                  preferred_element_type=jnp.float32)
    a = jnp.maximum(acc + b_ref[...], 0.0).astype(jnp.bfloat16)
    o_ref[...] = jnp.transpose(a.reshape(bb, m1, 32), (1, 0, 2))


def _tail_kernel(a_ref, w2_ref, b2_ref, w3_ref, b3_ref, w4_ref, b4_ref,
                 wf1_ref, bf1_ref, wf2_ref, bf2_ref, wf3_ref, bf3_ref,
                 o_ref):
    f32 = jnp.float32
    bb = a_ref.shape[1]

    a = a_ref[...].reshape(10, 2, 10, 2, bb, 32)
    acc = None
    for ki in range(4):
        for kj in range(4):
            g = a[ki // 2:ki // 2 + 9, ki % 2, kj // 2:kj // 2 + 9, kj % 2]
            c = jnp.dot(g.reshape(81 * bb, 32), w2_ref[ki * 4 + kj],
                        preferred_element_type=f32)
            acc = c if acc is None else acc + c
    a2 = jnp.maximum(acc + b2_ref[...], 0.0).astype(jnp.bfloat16)
    a2 = a2.reshape(9, 9, bb, 64)

    acc = None
    for ki in range(3):
        for kj in range(3):
            g = a2[ki:ki + 7, kj:kj + 7]
            c = jnp.dot(g.reshape(49 * bb, 64), w3_ref[ki * 3 + kj],
                        preferred_element_type=f32)
            acc = c if acc is None else acc + c
    a3 = jnp.maximum(acc + b3_ref[...], 0.0).astype(jnp.bfloat16)
    a3 = a3.reshape(7, 7, bb, 64)

    acc = None
    for ki in range(3):
        for kj in range(3):
            g = a3[ki:ki + 5, kj:kj + 5]
            c = jnp.dot(g.reshape(25 * bb, 64), w4_ref[ki * 3 + kj],
                        preferred_element_type=f32)
            acc = c if acc is None else acc + c
    a4 = jnp.maximum(acc + b4_ref[...], 0.0).astype(jnp.bfloat16)
    a4 = a4.reshape(25, bb, 32)

    acc = None
    for p in range(25):
        c = jnp.dot(a4[p], wf1_ref[p], preferred_element_type=f32)
        acc = c if acc is None else acc + c
    h = jnp.maximum(acc + bf1_ref[...], 0.0).astype(jnp.bfloat16)
    h = jnp.dot(h, wf2_ref[...], preferred_element_type=f32) + bf2_ref[...]
    h = jnp.maximum(h, 0.0).astype(jnp.bfloat16)
    q = jnp.dot(h, wf3_ref[...], preferred_element_type=f32) + bf3_ref[...]
    o_ref[...] = q


def _const_specs(arrs):
    specs = []
    for a in arrs:
        nd = a.ndim
        specs.append(pl.BlockSpec(a.shape, lambda i, _nd=nd: (0,) * _nd))
    return specs


def kernel(x, conv1_w, conv1_b, conv2_s, conv2_w, conv2_b,
           conv3_s, conv3_w, conv3_b, conv4_s, conv4_w, conv4_b,
           fc1_w, fc1_b, fc2_w, fc2_b, fc3_w, fc3_b):
    n = x.shape[0]
    xh = jnp.transpose(x, (0, 2, 3, 1)).astype(jnp.bfloat16)
    p1 = _im2col_nhwc(xh, 8, 8, 4)

    bc = 32 if n % 32 == 0 else n
    a1 = pl.pallas_call(
        _conv1_kernel,
        out_shape=jax.ShapeDtypeStruct((400, n, 32), jnp.bfloat16),
        grid=(n // bc,),
        in_specs=[pl.BlockSpec((bc, 400, 256), lambda i: (i, 0, 0))]
        + _const_specs([conv1_w, conv1_b]),
        out_specs=pl.BlockSpec((400, bc, 32), lambda i: (0, i, 0)),
        compiler_params=pltpu.CompilerParams(
            dimension_semantics=("arbitrary",)),
    )(p1, conv1_w, conv1_b)

    bb = 32 if n % 32 == 0 else n
    consts = [conv2_w, conv2_b, conv3_w, conv3_b, conv4_w, conv4_b,
              fc1_w, fc1_b, fc2_w, fc2_b, fc3_w, fc3_b]
    q = pl.pallas_call(
        _tail_kernel,
        out_shape=jax.ShapeDtypeStruct((n, 128), jnp.float32),
        grid=(n // bb,),
        in_specs=[pl.BlockSpec((400, bb, 32), lambda i: (0, i, 0))]
        + _const_specs(consts),
        out_specs=pl.BlockSpec((bb, 128), lambda i: (i, 0)),
        compiler_params=pltpu.CompilerParams(
            dimension_semantics=("arbitrary",)),
    )(a1, *consts)
    return q[:, :6]


# speedup vs baseline: 33.1496x; 1.3344x over previous
import jax
import jax.numpy as jnp
from jax.experimental import pallas as pl
from jax.experimental.pallas import tpu as pltpu


def _im2col_nchw(x, k, stride):
    n, c, h, w = x.shape
    ho = (h - k) // stride + 1
    wo = (w - k) // stride + 1
    taps = []
    for i in range(k):
        for j in range(k):
            taps.append(x[:, :, i:i + stride * ho:stride, j:j + stride * wo:stride])
    cols = jnp.stack(taps, axis=2)
    return jnp.transpose(cols.reshape(n, c * k * k, ho * wo), (0, 2, 1))


def _conv1_kernel(p_ref, w_ref, b_ref, o_ref):
    bb, m1, k1 = p_ref.shape
    acc = jnp.dot(p_ref[...].reshape(bb * m1, k1), w_ref[...],
                  preferred_element_type=jnp.float32)
    a = jnp.maximum(acc + b_ref[...], 0.0).astype(jnp.bfloat16)
    o_ref[...] = jnp.transpose(a.reshape(bb, m1, 32), (1, 0, 2))


def _tail_kernel(a_ref, w2_ref, b2_ref, w3_ref, b3_ref, w4_ref, b4_ref,
                 wf1_ref, bf1_ref, wf2_ref, bf2_ref, wf3_ref, bf3_ref,
                 o_ref):
    f32 = jnp.float32
    bb = a_ref.shape[1]

    a = a_ref[...].reshape(10, 2, 10, 2, bb, 32)
    acc = None
    for ki in range(4):
        for kj in range(4):
            g = a[ki // 2:ki // 2 + 9, ki % 2, kj // 2:kj // 2 + 9, kj % 2]
            c = jnp.dot(g.reshape(81 * bb, 32), w2_ref[ki * 4 + kj],
                        preferred_element_type=f32)
            acc = c if acc is None else acc + c
    a2 = jnp.maximum(acc + b2_ref[...], 0.0).astype(jnp.bfloat16)
    a2 = a2.reshape(9, 9, bb, 64)

    acc = None
    for ki in range(3):
        for kj in range(3):
            g = a2[ki:ki + 7, kj:kj + 7]
            c = jnp.dot(g.reshape(49 * bb, 64), w3_ref[ki * 3 + kj],
                        preferred_element_type=f32)
            acc = c if acc is None else acc + c
    a3 = jnp.maximum(acc + b3_ref[...], 0.0).astype(jnp.bfloat16)
    a3 = a3.reshape(7, 7, bb, 64)

    acc = None
    for ki in range(3):
        for kj in range(3):
            g = a3[ki:ki + 5, kj:kj + 5]
            c = jnp.dot(g.reshape(25 * bb, 64), w4_ref[ki * 3 + kj],
                        preferred_element_type=f32)
            acc = c if acc is None else acc + c
    a4 = jnp.maximum(acc + b4_ref[...], 0.0).astype(jnp.bfloat16)
    a4 = a4.reshape(25, bb, 32)

    acc = None
    for p in range(25):
        c = jnp.dot(a4[p], wf1_ref[p], preferred_element_type=f32)
        acc = c if acc is None else acc + c
    h = jnp.maximum(acc + bf1_ref[...], 0.0).astype(jnp.bfloat16)
    h = jnp.dot(h, wf2_ref[...], preferred_element_type=f32) + bf2_ref[...]
    h = jnp.maximum(h, 0.0).astype(jnp.bfloat16)
    q = jnp.dot(h, wf3_ref[...], preferred_element_type=f32) + bf3_ref[...]
    o_ref[...] = q


def _const_specs(arrs):
    specs = []
    for a in arrs:
        nd = a.ndim
        specs.append(pl.BlockSpec(a.shape, lambda i, _nd=nd: (0,) * _nd))
    return specs


def kernel(x, conv1_w, conv1_b, conv2_s, conv2_w, conv2_b,
           conv3_s, conv3_w, conv3_b, conv4_s, conv4_w, conv4_b,
           fc1_w, fc1_b, fc2_w, fc2_b, fc3_w, fc3_b):
    n = x.shape[0]
    p1 = _im2col_nchw(x.astype(jnp.bfloat16), 8, 4)
    w1 = jnp.transpose(conv1_w.reshape(8, 8, 4, 32),
                       (2, 0, 1, 3)).reshape(256, 32)

    bc = 32 if n % 32 == 0 else n
    a1 = pl.pallas_call(
        _conv1_kernel,
        out_shape=jax.ShapeDtypeStruct((400, n, 32), jnp.bfloat16),
        grid=(n // bc,),
        in_specs=[pl.BlockSpec((bc, 400, 256), lambda i: (i, 0, 0))]
        + _const_specs([w1, conv1_b]),
        out_specs=pl.BlockSpec((400, bc, 32), lambda i: (0, i, 0)),
        compiler_params=pltpu.CompilerParams(
            dimension_semantics=("arbitrary",)),
    )(p1, w1, conv1_b)

    bb = 32 if n % 32 == 0 else n
    consts = [conv2_w, conv2_b, conv3_w, conv3_b, conv4_w, conv4_b,
              fc1_w, fc1_b, fc2_w, fc2_b, fc3_w, fc3_b]
    q = pl.pallas_call(
        _tail_kernel,
        out_shape=jax.ShapeDtypeStruct((n, 128), jnp.float32),
        grid=(n // bb,),
        in_specs=[pl.BlockSpec((400, bb, 32), lambda i: (0, i, 0))]
        + _const_specs(consts),
        out_specs=pl.BlockSpec((bb, 128), lambda i: (i, 0)),
        compiler_params=pltpu.CompilerParams(
            dimension_semantics=("arbitrary",)),
    )(a1, *consts)
    return q[:, :6]


# speedup vs baseline: 44.7788x; 1.3508x over previous
import jax
import jax.numpy as jnp
from jax.experimental import pallas as pl
from jax.experimental.pallas import tpu as pltpu


def _full_kernel(p_ref, w1_ref, b1_ref, w2_ref, b2_ref, w3_ref, b3_ref,
                 w4_ref, b4_ref, wf1_ref, bf1_ref, wf2_ref, bf2_ref,
                 wf3_ref, bf3_ref, o_ref):
    f32 = jnp.float32
    bb = p_ref.shape[0]

    a = jnp.transpose(p_ref[...], (1, 0, 2)).reshape(21, 21, bb, 64)
    acc = None
    for ai in range(2):
        for aj in range(2):
            g = a[ai:ai + 20, aj:aj + 20]
            c = jnp.dot(g.reshape(400 * bb, 64), w1_ref[ai * 2 + aj],
                        preferred_element_type=f32)
            acc = c if acc is None else acc + c
    a1 = jnp.maximum(acc + b1_ref[...], 0.0).astype(jnp.bfloat16)

    a = a1.reshape(10, 2, 10, 2, bb, 32)
    acc = None
    for ki in range(4):
        for kj in range(4):
            g = a[ki // 2:ki // 2 + 9, ki % 2, kj // 2:kj // 2 + 9, kj % 2]
            c = jnp.dot(g.reshape(81 * bb, 32), w2_ref[ki * 4 + kj],
                        preferred_element_type=f32)
            acc = c if acc is None else acc + c
    a2 = jnp.maximum(acc + b2_ref[...], 0.0).astype(jnp.bfloat16)
    a2 = a2.reshape(9, 9, bb, 64)

    acc = None
    for ki in range(3):
        for kj in range(3):
            g = a2[ki:ki + 7, kj:kj + 7]
            c = jnp.dot(g.reshape(49 * bb, 64), w3_ref[ki * 3 + kj],
                        preferred_element_type=f32)
            acc = c if acc is None else acc + c
    a3 = jnp.maximum(acc + b3_ref[...], 0.0).astype(jnp.bfloat16)
    a3 = a3.reshape(7, 7, bb, 64)

    acc = None
    for ki in range(3):
        for kj in range(3):
            g = a3[ki:ki + 5, kj:kj + 5]
            c = jnp.dot(g.reshape(25 * bb, 64), w4_ref[ki * 3 + kj],
                        preferred_element_type=f32)
            acc = c if acc is None else acc + c
    a4 = jnp.maximum(acc + b4_ref[...], 0.0).astype(jnp.bfloat16)
    a4 = a4.reshape(25, bb, 32)

    acc = None
    for p in range(25):
        c = jnp.dot(a4[p], wf1_ref[p], preferred_element_type=f32)
        acc = c if acc is None else acc + c
    h = jnp.maximum(acc + bf1_ref[...], 0.0).astype(jnp.bfloat16)
    h = jnp.dot(h, wf2_ref[...], preferred_element_type=f32) + bf2_ref[...]
    h = jnp.maximum(h, 0.0).astype(jnp.bfloat16)
    q = jnp.dot(h, wf3_ref[...], preferred_element_type=f32) + bf3_ref[...]
    o_ref[...] = q


def _const_specs(arrs):
    specs = []
    for a in arrs:
        nd = a.ndim
        specs.append(pl.BlockSpec(a.shape, lambda i, _nd=nd: (0,) * _nd))
    return specs


def kernel(x, conv1_w, conv1_b, conv2_s, conv2_w, conv2_b,
           conv3_s, conv3_w, conv3_b, conv4_s, conv4_w, conv4_b,
           fc1_w, fc1_b, fc2_w, fc2_b, fc3_w, fc3_b):
    n = x.shape[0]
    p0 = jnp.transpose(
        x.astype(jnp.bfloat16).reshape(n, 4, 21, 4, 21, 4),
        (0, 2, 4, 1, 3, 5)).reshape(n, 441, 64)
    w1 = jnp.transpose(conv1_w.reshape(2, 4, 2, 4, 4, 32),
                       (0, 2, 4, 1, 3, 5)).reshape(4, 64, 32)

    bb = 32 if n % 32 == 0 else n
    consts = [w1, conv1_b, conv2_w, conv2_b, conv3_w, conv3_b,
              conv4_w, conv4_b, fc1_w, fc1_b, fc2_w, fc2_b, fc3_w, fc3_b]
    q = pl.pallas_call(
        _full_kernel,
        out_shape=jax.ShapeDtypeStruct((n, 128), jnp.float32),
        grid=(n // bb,),
        in_specs=[pl.BlockSpec((bb, 441, 64), lambda i: (i, 0, 0))]
        + _const_specs(consts),
        out_specs=pl.BlockSpec((bb, 128), lambda i: (i, 0)),
        compiler_params=pltpu.CompilerParams(
            dimension_semantics=("arbitrary",)),
    )(p0, *consts)
    return q[:, :6]


# speedup vs baseline: 47.4337x; 1.0593x over previous
import jax
import jax.numpy as jnp
from jax.experimental import pallas as pl
from jax.experimental.pallas import tpu as pltpu


def _full_kernel(p_ref, w1_ref, b1_ref, w2_ref, b2_ref, w3_ref, b3_ref,
                 w4_ref, b4_ref, wf1_ref, bf1_ref, wf2_ref, bf2_ref,
                 wf3_ref, bf3_ref, o_ref):
    f32 = jnp.float32
    bb = p_ref.shape[1]

    a = p_ref[...].reshape(21, 21, bb, 64)
    acc = None
    for ai in range(2):
        for aj in range(2):
            g = a[ai:ai + 20, aj:aj + 20]
            c = jnp.dot(g.reshape(400 * bb, 64), w1_ref[ai * 2 + aj],
                        preferred_element_type=f32)
            acc = c if acc is None else acc + c
    a1 = jnp.maximum(acc + b1_ref[...], 0.0).astype(jnp.bfloat16)

    a = a1.reshape(10, 2, 10, 2, bb, 32)
    acc = None
    for ki in range(4):
        for kj in range(4):
            g = a[ki // 2:ki // 2 + 9, ki % 2, kj // 2:kj // 2 + 9, kj % 2]
            c = jnp.dot(g.reshape(81 * bb, 32), w2_ref[ki * 4 + kj],
                        preferred_element_type=f32)
            acc = c if acc is None else acc + c
    a2 = jnp.maximum(acc + b2_ref[...], 0.0).astype(jnp.bfloat16)
    a2 = a2.reshape(9, 9, bb, 64)

    acc = None
    for ki in range(3):
        for kj in range(3):
            g = a2[ki:ki + 7, kj:kj + 7]
            c = jnp.dot(g.reshape(49 * bb, 64), w3_ref[ki * 3 + kj],
                        preferred_element_type=f32)
            acc = c if acc is None else acc + c
    a3 = jnp.maximum(acc + b3_ref[...], 0.0).astype(jnp.bfloat16)
    a3 = a3.reshape(7, 7, bb, 64)

    acc = None
    for ki in range(3):
        for kj in range(3):
            g = a3[ki:ki + 5, kj:kj + 5]
            c = jnp.dot(g.reshape(25 * bb, 64), w4_ref[ki * 3 + kj],
                        preferred_element_type=f32)
            acc = c if acc is None else acc + c
    a4 = jnp.maximum(acc + b4_ref[...], 0.0).astype(jnp.bfloat16)
    a4 = a4.reshape(25, bb, 32)

    acc = None
    for p in range(25):
        c = jnp.dot(a4[p], wf1_ref[p], preferred_element_type=f32)
        acc = c if acc is None else acc + c
    h = jnp.maximum(acc + bf1_ref[...], 0.0).astype(jnp.bfloat16)
    h = jnp.dot(h, wf2_ref[...], preferred_element_type=f32) + bf2_ref[...]
    h = jnp.maximum(h, 0.0).astype(jnp.bfloat16)
    q = jnp.dot(h, wf3_ref[...], preferred_element_type=f32) + bf3_ref[...]
    o_ref[...] = q


def _const_specs(arrs):
    specs = []
    for a in arrs:
        nd = a.ndim
        specs.append(pl.BlockSpec(a.shape, lambda i, _nd=nd: (0,) * _nd))
    return specs


def kernel(x, conv1_w, conv1_b, conv2_s, conv2_w, conv2_b,
           conv3_s, conv3_w, conv3_b, conv4_s, conv4_w, conv4_b,
           fc1_w, fc1_b, fc2_w, fc2_b, fc3_w, fc3_b):
    n = x.shape[0]
    p0 = jnp.transpose(
        x.astype(jnp.bfloat16).reshape(n, 4, 21, 4, 21, 4),
        (2, 4, 0, 1, 3, 5)).reshape(441, n, 64)
    w1 = jnp.transpose(conv1_w.reshape(2, 4, 2, 4, 4, 32),
                       (0, 2, 4, 1, 3, 5)).reshape(4, 64, 32)

    bb = 32 if n % 32 == 0 else n
    consts = [w1, conv1_b, conv2_w, conv2_b, conv3_w, conv3_b,
              conv4_w, conv4_b, fc1_w, fc1_b, fc2_w, fc2_b, fc3_w, fc3_b]
    q = pl.pallas_call(
        _full_kernel,
        out_shape=jax.ShapeDtypeStruct((n, 128), jnp.float32),
        grid=(n // bb,),
        in_specs=[pl.BlockSpec((441, bb, 64), lambda i: (0, i, 0))]
        + _const_specs(consts),
        out_specs=pl.BlockSpec((bb, 128), lambda i: (i, 0)),
        compiler_params=pltpu.CompilerParams(
            dimension_semantics=("arbitrary",)),
    )(p0, *consts)
    return q[:, :6]


# speedup vs baseline: 67.5661x; 1.4244x over previous
import jax
import jax.numpy as jnp
from jax.experimental import pallas as pl
from jax.experimental.pallas import tpu as pltpu


def _full_kernel(p_ref, w1_ref, b1_ref, w2_ref, b2_ref, w3_ref, b3_ref,
                 w4_ref, b4_ref, wf1_ref, bf1_ref, wf2_ref, bf2_ref,
                 wf3_ref, bf3_ref, o_ref):
    f32 = jnp.float32
    bb = p_ref.shape[1]

    a = p_ref[...].reshape(21, 21, bb, 64)
    acc = None
    for ai in range(2):
        for aj in range(2):
            g = a[ai:ai + 20, aj:aj + 20]
            c = jnp.dot(g.reshape(400 * bb, 64), w1_ref[ai * 2 + aj],
                        preferred_element_type=f32)
            acc = c if acc is None else acc + c
    a1 = jnp.maximum(acc + b1_ref[...], 0.0).astype(jnp.bfloat16)

    a = a1.reshape(10, 2, 10, 2, bb, 32)
    acc = None
    for ki in range(4):
        for kj in range(4):
            g = a[ki // 2:ki // 2 + 9, ki % 2, kj // 2:kj // 2 + 9, kj % 2]
            c = jnp.dot(g.reshape(81 * bb, 32), w2_ref[ki * 4 + kj],
                        preferred_element_type=f32)
            acc = c if acc is None else acc + c
    a2 = jnp.maximum(acc + b2_ref[...], 0.0).astype(jnp.bfloat16)
    a2 = a2.reshape(9, 9, bb, 64)

    acc = None
    for ki in range(3):
        for kj in range(3):
            g = a2[ki:ki + 7, kj:kj + 7]
            c = jnp.dot(g.reshape(49 * bb, 64), w3_ref[ki * 3 + kj],
                        preferred_element_type=f32)
            acc = c if acc is None else acc + c
    a3 = jnp.maximum(acc + b3_ref[...], 0.0).astype(jnp.bfloat16)
    a3 = a3.reshape(7, 7, bb, 64)

    acc = None
    for ki in range(3):
        for kj in range(3):
            g = a3[ki:ki + 5, kj:kj + 5]
            c = jnp.dot(g.reshape(25 * bb, 64), w4_ref[ki * 3 + kj],
                        preferred_element_type=f32)
            acc = c if acc is None else acc + c
    a4 = jnp.maximum(acc + b4_ref[...], 0.0).astype(jnp.bfloat16)
    a4 = a4.reshape(25, bb, 32)

    acc = None
    for p in range(25):
        c = jnp.dot(a4[p], wf1_ref[p], preferred_element_type=f32)
        acc = c if acc is None else acc + c
    h = jnp.maximum(acc + bf1_ref[...], 0.0).astype(jnp.bfloat16)
    h = jnp.dot(h, wf2_ref[...], preferred_element_type=f32) + bf2_ref[...]
    h = jnp.maximum(h, 0.0).astype(jnp.bfloat16)
    q = jnp.dot(h, wf3_ref[...], preferred_element_type=f32) + bf3_ref[...]
    o_ref[...] = q


def _const_specs(arrs):
    specs = []
    for a in arrs:
        nd = a.ndim
        specs.append(pl.BlockSpec(a.shape, lambda i, _nd=nd: (0,) * _nd))
    return specs


def kernel(x, conv1_w, conv1_b, conv2_s, conv2_w, conv2_b,
           conv3_s, conv3_w, conv3_b, conv4_s, conv4_w, conv4_b,
           fc1_w, fc1_b, fc2_w, fc2_b, fc3_w, fc3_b):
    n = x.shape[0]
    p0 = jnp.transpose(
        x.astype(jnp.bfloat16).reshape(n, 4, 21, 4, 21, 4),
        (2, 4, 0, 1, 3, 5)).reshape(441, n, 64)
    w1 = jnp.transpose(conv1_w.reshape(2, 4, 2, 4, 4, 32),
                       (0, 2, 4, 1, 3, 5)).reshape(4, 64, 32)

    bb = 64 if n % 64 == 0 else n
    consts = [w1, conv1_b, conv2_w, conv2_b, conv3_w, conv3_b,
              conv4_w, conv4_b, fc1_w, fc1_b, fc2_w, fc2_b, fc3_w, fc3_b]
    q = pl.pallas_call(
        _full_kernel,
        out_shape=jax.ShapeDtypeStruct((n, 128), jnp.float32),
        grid=(n // bb,),
        in_specs=[pl.BlockSpec((441, bb, 64), lambda i: (0, i, 0))]
        + _const_specs(consts),
        out_specs=pl.BlockSpec((bb, 128), lambda i: (i, 0)),
        compiler_params=pltpu.CompilerParams(
            dimension_semantics=("arbitrary",)),
    )(p0, *consts)
    return q[:, :6]


# speedup vs baseline: 67.6495x; 1.0012x over previous
import jax
import jax.numpy as jnp
from jax.experimental import pallas as pl
from jax.experimental.pallas import tpu as pltpu


def _full_kernel(p_ref, w1_ref, b1_ref, w2_ref, b2_ref, w3_ref, b3_ref,
                 w4_ref, b4_ref, wf1_ref, bf1_ref, wf2_ref, bf2_ref,
                 wf3_ref, bf3_ref, o_ref):
    f32 = jnp.float32
    bb = p_ref.shape[1]

    a = p_ref[...].reshape(21, 21, bb, 64)
    acc = None
    for ai in range(2):
        g = jnp.concatenate(
            [a[ai:ai + 20, aj:aj + 20].reshape(400 * bb, 64)
             for aj in range(2)], axis=-1)
        c = jnp.dot(g, w1_ref[ai], preferred_element_type=f32)
        acc = c if acc is None else acc + c
    a1 = jnp.maximum(acc + b1_ref[...], 0.0).astype(jnp.bfloat16)

    a = a1.reshape(10, 2, 10, 2, bb, 32)
    acc = None
    for ki in range(4):
        g = jnp.concatenate(
            [a[ki // 2:ki // 2 + 9, ki % 2,
               kj // 2:kj // 2 + 9, kj % 2].reshape(81 * bb, 32)
             for kj in range(4)], axis=-1)
        c = jnp.dot(g, w2_ref[ki], preferred_element_type=f32)
        acc = c if acc is None else acc + c
    a2 = jnp.maximum(acc + b2_ref[...], 0.0).astype(jnp.bfloat16)
    a2 = a2.reshape(9, 9, bb, 64)

    acc = None
    for ki in range(3):
        g = jnp.concatenate(
            [a2[ki:ki + 7, kj:kj + 7].reshape(49 * bb, 64)
             for kj in range(3)], axis=-1)
        c = jnp.dot(g, w3_ref[ki], preferred_element_type=f32)
        acc = c if acc is None else acc + c
    a3 = jnp.maximum(acc + b3_ref[...], 0.0).astype(jnp.bfloat16)
    a3 = a3.reshape(7, 7, bb, 64)

    acc = None
    for ki in range(3):
        g = jnp.concatenate(
            [a3[ki:ki + 5, kj:kj + 5].reshape(25 * bb, 64)
             for kj in range(3)], axis=-1)
        c = jnp.dot(g, w4_ref[ki], preferred_element_type=f32)
        acc = c if acc is None else acc + c
    a4 = jnp.maximum(acc + b4_ref[...], 0.0).astype(jnp.bfloat16)
    a4 = a4.reshape(25, bb, 32)

    acc = None
    for p in range(5):
        g = jnp.concatenate([a4[5 * p + q] for q in range(5)], axis=-1)
        c = jnp.dot(g, wf1_ref[p], preferred_element_type=f32)
        acc = c if acc is None else acc + c
    h = jnp.maximum(acc + bf1_ref[...], 0.0).astype(jnp.bfloat16)
    h = jnp.dot(h, wf2_ref[...], preferred_element_type=f32) + bf2_ref[...]
    h = jnp.maximum(h, 0.0).astype(jnp.bfloat16)
    q = jnp.dot(h, wf3_ref[...], preferred_element_type=f32) + bf3_ref[...]
    o_ref[...] = q


def _const_specs(arrs):
    specs = []
    for a in arrs:
        nd = a.ndim
        specs.append(pl.BlockSpec(a.shape, lambda i, _nd=nd: (0,) * _nd))
    return specs


def kernel(x, conv1_w, conv1_b, conv2_s, conv2_w, conv2_b,
           conv3_s, conv3_w, conv3_b, conv4_s, conv4_w, conv4_b,
           fc1_w, fc1_b, fc2_w, fc2_b, fc3_w, fc3_b):
    n = x.shape[0]
    p0 = jnp.transpose(
        x.astype(jnp.bfloat16).reshape(n, 4, 21, 4, 21, 4),
        (2, 4, 0, 1, 3, 5)).reshape(441, n, 64)
    w1 = jnp.transpose(conv1_w.reshape(2, 4, 2, 4, 4, 32),
                       (0, 2, 4, 1, 3, 5)).reshape(2, 128, 32)

    bb = 64 if n % 64 == 0 else n
    consts = [w1, conv1_b,
              conv2_w.reshape(4, 128, 64), conv2_b,
              conv3_w.reshape(3, 192, 64), conv3_b,
              conv4_w.reshape(3, 192, 32), conv4_b,
              fc1_w.reshape(5, 160, 512), fc1_b,
              fc2_w, fc2_b, fc3_w, fc3_b]
    q = pl.pallas_call(
        _full_kernel,
        out_shape=jax.ShapeDtypeStruct((n, 128), jnp.float32),
        grid=(n // bb,),
        in_specs=[pl.BlockSpec((441, bb, 64), lambda i: (0, i, 0))]
        + _const_specs(consts),
        out_specs=pl.BlockSpec((bb, 128), lambda i: (i, 0)),
        compiler_params=pltpu.CompilerParams(
            dimension_semantics=("arbitrary",)),
    )(p0, *consts)
    return q[:, :6]


# speedup vs baseline: 71.8575x; 1.0622x over previous
import jax
import jax.numpy as jnp
from jax.experimental import pallas as pl
from jax.experimental.pallas import tpu as pltpu


def _full_kernel(p_ref, w1_ref, b1_ref, w2_ref, b2_ref, w3_ref, b3_ref,
                 w4_ref, b4_ref, wf1_ref, bf1_ref, wf2_ref, bf2_ref,
                 wf3_ref, bf3_ref, o_ref):
    f32 = jnp.float32
    bb = p_ref.shape[1]

    a = p_ref[...].reshape(21, 21, bb, 64)
    acc = None
    for ai in range(2):
        g = jnp.concatenate(
            [a[ai:ai + 20, aj:aj + 20].reshape(400 * bb, 64)
             for aj in range(2)], axis=-1)
        c = jnp.dot(g, w1_ref[ai], preferred_element_type=f32)
        acc = c if acc is None else acc + c
    a1 = jnp.maximum(acc + b1_ref[...], 0.0).astype(jnp.bfloat16)

    a = a1.reshape(10, 2, 10, 2, bb, 32)
    acc = None
    for ki in range(4):
        g = jnp.concatenate(
            [a[ki // 2:ki // 2 + 9, ki % 2,
               kj // 2:kj // 2 + 9, kj % 2].reshape(81 * bb, 32)
             for kj in range(4)], axis=-1)
        c = jnp.dot(g, w2_ref[ki], preferred_element_type=f32)
        acc = c if acc is None else acc + c
    a2 = jnp.maximum(acc + b2_ref[...], 0.0).astype(jnp.bfloat16)
    a2 = a2.reshape(9, 9, bb, 64)

    acc = None
    for ki in range(3):
        g = jnp.concatenate(
            [a2[ki:ki + 7, kj:kj + 7].reshape(49 * bb, 64)
             for kj in range(3)], axis=-1)
        c = jnp.dot(g, w3_ref[ki], preferred_element_type=f32)
        acc = c if acc is None else acc + c
    a3 = jnp.maximum(acc + b3_ref[...], 0.0).astype(jnp.bfloat16)
    a3 = a3.reshape(7, 7, bb, 64)

    acc = None
    for ki in range(3):
        g = jnp.concatenate(
            [a3[ki:ki + 5, kj:kj + 5].reshape(25 * bb, 64)
             for kj in range(3)], axis=-1)
        c = jnp.dot(g, w4_ref[ki], preferred_element_type=f32)
        acc = c if acc is None else acc + c
    a4 = jnp.maximum(acc + b4_ref[...], 0.0).astype(jnp.bfloat16)
    a4 = a4.reshape(25, bb, 32)

    acc = None
    for p in range(5):
        g = jnp.concatenate([a4[5 * p + q] for q in range(5)], axis=-1)
        c = jnp.dot(g, wf1_ref[p], preferred_element_type=f32)
        acc = c if acc is None else acc + c
    h = jnp.maximum(acc + bf1_ref[...], 0.0).astype(jnp.bfloat16)
    h = jnp.dot(h, wf2_ref[...], preferred_element_type=f32) + bf2_ref[...]
    h = jnp.maximum(h, 0.0).astype(jnp.bfloat16)
    q = jnp.dot(h, wf3_ref[...], preferred_element_type=f32) + bf3_ref[...]
    o_ref[...] = q


def _const_specs(arrs):
    specs = []
    for a in arrs:
        nd = a.ndim
        specs.append(pl.BlockSpec(a.shape, lambda i, _nd=nd: (0,) * _nd))
    return specs


def _run_half(xh, consts):
    n = xh.shape[0]
    p0 = jnp.transpose(
        xh.astype(jnp.bfloat16).reshape(n, 4, 21, 4, 21, 4),
        (2, 4, 0, 1, 3, 5)).reshape(441, n, 64)
    bb = 64 if n % 64 == 0 else n
    return pl.pallas_call(
        _full_kernel,
        out_shape=jax.ShapeDtypeStruct((n, 128), jnp.float32),
        grid=(n // bb,),
        in_specs=[pl.BlockSpec((441, bb, 64), lambda i: (0, i, 0))]
        + _const_specs(consts),
        out_specs=pl.BlockSpec((bb, 128), lambda i: (i, 0)),
        compiler_params=pltpu.CompilerParams(
            dimension_semantics=("arbitrary",)),
    )(p0, *consts)


def kernel(x, conv1_w, conv1_b, conv2_s, conv2_w, conv2_b,
           conv3_s, conv3_w, conv3_b, conv4_s, conv4_w, conv4_b,
           fc1_w, fc1_b, fc2_w, fc2_b, fc3_w, fc3_b):
    n = x.shape[0]
    w1 = jnp.transpose(conv1_w.reshape(2, 4, 2, 4, 4, 32),
                       (0, 2, 4, 1, 3, 5)).reshape(2, 128, 32)
    consts = [w1, conv1_b,
              conv2_w.reshape(4, 128, 64), conv2_b,
              conv3_w.reshape(3, 192, 64), conv3_b,
              conv4_w.reshape(3, 192, 32), conv4_b,
              fc1_w.reshape(5, 160, 512), fc1_b,
              fc2_w, fc2_b, fc3_w, fc3_b]
    if n % 128 == 0:
        q = jnp.concatenate([_run_half(x[:n // 2], consts),
                             _run_half(x[n // 2:], consts)], axis=0)
    else:
        q = _run_half(x, consts)
    return q[:, :6]


# speedup vs baseline: 73.1082x; 1.0174x over previous
import jax
import jax.numpy as jnp
from jax.experimental import pallas as pl
from jax.experimental.pallas import tpu as pltpu


def _full_kernel(p_ref, w1_ref, b1_ref, w2_ref, b2_ref, w3_ref, b3_ref,
                 w4_ref, b4_ref, wf1_ref, bf1_ref, wf2_ref, bf2_ref,
                 wf3_ref, bf3_ref, o_ref):
    f32 = jnp.float32
    bb = p_ref.shape[1]

    a = p_ref[...].reshape(21, 21, bb, 64)
    g = jnp.concatenate(
        [a[ai:ai + 20, aj:aj + 20].reshape(400 * bb, 64)
         for ai in range(2) for aj in range(2)], axis=-1)
    acc = jnp.dot(g, w1_ref[...], preferred_element_type=f32)
    a1 = jnp.maximum(acc + b1_ref[...], 0.0).astype(jnp.bfloat16)

    a = a1.reshape(10, 2, 10, 2, bb, 32)
    g = jnp.concatenate(
        [a[ki // 2:ki // 2 + 9, ki % 2,
           kj // 2:kj // 2 + 9, kj % 2].reshape(81 * bb, 32)
         for ki in range(4) for kj in range(4)], axis=-1)
    acc = jnp.dot(g, w2_ref[...], preferred_element_type=f32)
    a2 = jnp.maximum(acc + b2_ref[...], 0.0).astype(jnp.bfloat16)
    a2 = a2.reshape(9, 9, bb, 64)

    acc = None
    for ki in range(3):
        g = jnp.concatenate(
            [a2[ki:ki + 7, kj:kj + 7].reshape(49 * bb, 64)
             for kj in range(3)], axis=-1)
        c = jnp.dot(g, w3_ref[ki], preferred_element_type=f32)
        acc = c if acc is None else acc + c
    a3 = jnp.maximum(acc + b3_ref[...], 0.0).astype(jnp.bfloat16)
    a3 = a3.reshape(7, 7, bb, 64)

    acc = None
    for ki in range(3):
        g = jnp.concatenate(
            [a3[ki:ki + 5, kj:kj + 5].reshape(25 * bb, 64)
             for kj in range(3)], axis=-1)
        c = jnp.dot(g, w4_ref[ki], preferred_element_type=f32)
        acc = c if acc is None else acc + c
    a4 = jnp.maximum(acc + b4_ref[...], 0.0).astype(jnp.bfloat16)
    a4 = a4.reshape(25, bb, 32)

    acc = None
    for p in range(5):
        g = jnp.concatenate([a4[5 * p + q] for q in range(5)], axis=-1)
        c = jnp.dot(g, wf1_ref[p], preferred_element_type=f32)
        acc = c if acc is None else acc + c
    h = jnp.maximum(acc + bf1_ref[...], 0.0).astype(jnp.bfloat16)
    h = jnp.dot(h, wf2_ref[...], preferred_element_type=f32) + bf2_ref[...]
    h = jnp.maximum(h, 0.0).astype(jnp.bfloat16)
    q = jnp.dot(h, wf3_ref[...], preferred_element_type=f32) + bf3_ref[...]
    o_ref[...] = q


def _const_specs(arrs):
    specs = []
    for a in arrs:
        nd = a.ndim
        specs.append(pl.BlockSpec(a.shape, lambda i, _nd=nd: (0,) * _nd))
    return specs


def _run_half(xh, consts):
    n = xh.shape[0]
    p0 = jnp.transpose(
        xh.astype(jnp.bfloat16).reshape(n, 4, 21, 4, 21, 4),
        (2, 4, 0, 1, 3, 5)).reshape(441, n, 64)
    bb = 64 if n % 64 == 0 else n
    return pl.pallas_call(
        _full_kernel,
        out_shape=jax.ShapeDtypeStruct((n, 128), jnp.float32),
        grid=(n // bb,),
        in_specs=[pl.BlockSpec((441, bb, 64), lambda i: (0, i, 0))]
        + _const_specs(consts),
        out_specs=pl.BlockSpec((bb, 128), lambda i: (i, 0)),
        compiler_params=pltpu.CompilerParams(
            dimension_semantics=("arbitrary",)),
    )(p0, *consts)


def kernel(x, conv1_w, conv1_b, conv2_s, conv2_w, conv2_b,
           conv3_s, conv3_w, conv3_b, conv4_s, conv4_w, conv4_b,
           fc1_w, fc1_b, fc2_w, fc2_b, fc3_w, fc3_b):
    n = x.shape[0]
    w1 = jnp.transpose(conv1_w.reshape(2, 4, 2, 4, 4, 32),
                       (0, 2, 4, 1, 3, 5)).reshape(256, 32)
    consts = [w1, conv1_b,
              conv2_w.reshape(512, 64), conv2_b,
              conv3_w.reshape(3, 192, 64), conv3_b,
              conv4_w.reshape(3, 192, 32), conv4_b,
              fc1_w.reshape(5, 160, 512), fc1_b,
              fc2_w, fc2_b, fc3_w, fc3_b]
    if n % 128 == 0:
        q = jnp.concatenate([_run_half(x[:n // 2], consts),
                             _run_half(x[n // 2:], consts)], axis=0)
    else:
        q = _run_half(x, consts)
    return q[:, :6]
